# baseline (speedup 1.0000x reference)
import functools

import jax
import jax.numpy as jnp
from jax import lax
from jax.experimental import pallas as pl
from jax.experimental.pallas import tpu as pltpu
from jax.experimental.pallas import tpu_sc as plsc

B = 4096
L = 50
LP = 56
D = 300
DP = 384
NPAIR = 2 * B
NW = 32
PAIRS_PER_W = NPAIR // NW
OUT_SLAB = 128
_NCHUNK = 304 // 16

_PAD_ROWS = 2000
VOCAB = 100000


def _pad_body(in_ref, out_ref):
    out_ref[...] = jnp.concatenate(
        [in_ref[...], jnp.zeros((_PAD_ROWS, DP - D), jnp.float32)], axis=1)


_pad_table = pl.pallas_call(
    _pad_body,
    grid=(VOCAB // _PAD_ROWS,),
    in_specs=[pl.BlockSpec((_PAD_ROWS, D), lambda i: (i, 0))],
    out_specs=pl.BlockSpec((_PAD_ROWS, DP), lambda i: (i, 0)),
    out_shape=jax.ShapeDtypeStruct((VOCAB, DP), jnp.float32),
)


def _sc_body(idx_hbm, embed_hbm, out_hbm, idx_v, rows0, rows1, out_v,
             sem0, sem1):
    nc = 2
    wid = lax.axis_index("s") * nc + lax.axis_index("c")
    base = wid * PAIRS_PER_W
    pltpu.sync_copy(idx_hbm.at[pl.ds(base, PAIRS_PER_W)], idx_v)

    bufs = (rows0, rows1)
    sems = (sem0, sem1)

    def start(p, b):
        pltpu.async_copy(embed_hbm.at[idx_v.at[p]], bufs[b], sems[b])

    start(0, 0)
    start(1, 1)

    @pl.loop(0, PAIRS_PER_W // 2)
    def _grp(g):
        for b in range(2):
            p = 2 * g + b
            rows_v = bufs[b]
            pltpu.make_async_copy(
                embed_hbm.at[idx_v.at[p]], rows_v, sems[b]).wait()

            def row_max(i, acc, rows_v=rows_v):
                return tuple(
                    jnp.maximum(a, rows_v[i, pl.ds(16 * k, 16)])
                    for k, a in enumerate(acc)
                )

            acc0 = tuple(rows_v[0, pl.ds(16 * k, 16)] for k in range(_NCHUNK))
            acc = lax.fori_loop(1, LP, row_max, acc0)
            r = lax.rem(p, OUT_SLAB)
            for k, a in enumerate(acc):
                out_v[r, pl.ds(16 * k, 16)] = a

            @pl.when(p + 2 < PAIRS_PER_W)
            def _():
                start(p + 2, b)

            @pl.when(r == OUT_SLAB - 1)
            def _():
                slab = p // OUT_SLAB
                pltpu.sync_copy(
                    out_v,
                    out_hbm.at[pl.ds(base + slab * OUT_SLAB, OUT_SLAB)])


@functools.cache
def _gather_maxpool():
    return functools.partial(
        pl.kernel,
        out_type=jax.ShapeDtypeStruct((NPAIR, DP), jnp.float32),
        mesh=plsc.VectorSubcoreMesh(core_axis_name="c", subcore_axis_name="s"),
        scratch_types=[
            pltpu.VMEM((PAIRS_PER_W, LP), jnp.int32),
            pltpu.VMEM((LP, DP), jnp.float32),
            pltpu.VMEM((LP, DP), jnp.float32),
            pltpu.VMEM((OUT_SLAB, DP), jnp.float32),
            pltpu.SemaphoreType.DMA,
            pltpu.SemaphoreType.DMA,
        ],
        compiler_params=pltpu.CompilerParams(use_tc_tiling_on_sc=True),
    )(_sc_body)


def _head_body(pooled_ref, wc_ref, wr_ref, b_ref, lab_ref, loss_ref, logits_ref):
    dn = (((1,), (0,)), ((), ()))
    logits = (
        lax.dot_general(pooled_ref[0:B, 0:D], wc_ref[...], dn,
                        precision=lax.Precision.HIGHEST,
                        preferred_element_type=jnp.float32)
        + lax.dot_general(pooled_ref[B:NPAIR, 0:D], wr_ref[...], dn,
                          precision=lax.Precision.HIGHEST,
                          preferred_element_type=jnp.float32)
        + b_ref[...]
    )
    m = jnp.max(logits, axis=1, keepdims=True)
    e = jnp.exp(logits - m)
    lse = m + jnp.log(jnp.sum(e, axis=1, keepdims=True))
    picked = jnp.where(lab_ref[...] == 0, logits[:, 0:1], logits[:, 1:2])
    loss_ref[...] = jnp.sum(lse - picked, axis=0, keepdims=True) * (1.0 / B)
    logits_ref[...] = logits


_head = pl.pallas_call(
    _head_body,
    out_shape=(
        jax.ShapeDtypeStruct((1, 1), jnp.float32),
        jax.ShapeDtypeStruct((B, 2), jnp.float32),
    ),
)


def kernel(claims, reasons, label_ids, embed, W, b):
    idx = jnp.concatenate([claims, reasons], axis=0).astype(jnp.int32)
    idx = jnp.concatenate(
        [idx, jnp.broadcast_to(idx[:, :1], (NPAIR, LP - L))], axis=1)
    embed_p = jnp.pad(embed, ((0, 0), (0, DP - D)))
    pooled = _gather_maxpool()(idx, embed_p)
    loss, logits = _head(
        pooled,
        W[:D],
        W[D:],
        b.reshape(1, 2),
        label_ids.reshape(B, 1).astype(jnp.int32),
    )
    return loss.reshape(()), logits


# speedup vs baseline: 1.8651x; 1.8651x over previous
import functools

import jax
import jax.numpy as jnp
from jax import lax
from jax.experimental import pallas as pl
from jax.experimental.pallas import tpu as pltpu
from jax.experimental.pallas import tpu_sc as plsc

B = 4096
L = 50
LP = 56
D = 300
DP = 384
NPAIR = 2 * B
NW = 32
PAIRS_PER_W = NPAIR // NW
OUT_SLAB = 128
_NCHUNK = 304 // 16

_PAD_ROWS = 1024
VOCAB = 100000


def _pad_body(in_ref, out_ref):
    y = jnp.transpose(in_ref[...], (1, 0))
    out_ref[...] = jnp.concatenate(
        [y, jnp.zeros((_PAD_ROWS, DP - D), jnp.float32)], axis=1)


_pad_table = pl.pallas_call(
    _pad_body,
    grid=((VOCAB + _PAD_ROWS - 1) // _PAD_ROWS,),
    in_specs=[pl.BlockSpec((D, _PAD_ROWS), lambda i: (0, i))],
    out_specs=pl.BlockSpec((_PAD_ROWS, DP), lambda i: (i, 0)),
    out_shape=jax.ShapeDtypeStruct((VOCAB, DP), jnp.float32),
)


def _sc_body(idx_hbm, embed_hbm, out_hbm, idx_v, rows0, rows1, out_v,
             sem0, sem1):
    nc = 2
    wid = lax.axis_index("s") * nc + lax.axis_index("c")
    base = wid * PAIRS_PER_W
    pltpu.sync_copy(idx_hbm.at[pl.ds(base, PAIRS_PER_W)], idx_v)

    bufs = (rows0, rows1)
    sems = (sem0, sem1)

    def start(p, b):
        pltpu.async_copy(embed_hbm.at[idx_v.at[p]], bufs[b], sems[b])

    start(0, 0)
    start(1, 1)

    @pl.loop(0, PAIRS_PER_W // 2)
    def _grp(g):
        for b in range(2):
            p = 2 * g + b
            rows_v = bufs[b]
            pltpu.make_async_copy(
                embed_hbm.at[idx_v.at[p]], rows_v, sems[b]).wait()

            def row_max(i, acc, rows_v=rows_v):
                return tuple(
                    jnp.maximum(a, rows_v[i, pl.ds(16 * k, 16)])
                    for k, a in enumerate(acc)
                )

            acc0 = tuple(rows_v[0, pl.ds(16 * k, 16)] for k in range(_NCHUNK))
            acc = lax.fori_loop(1, LP, row_max, acc0)
            r = lax.rem(p, OUT_SLAB)
            for k, a in enumerate(acc):
                out_v[r, pl.ds(16 * k, 16)] = a

            @pl.when(p + 2 < PAIRS_PER_W)
            def _():
                start(p + 2, b)

            @pl.when(r == OUT_SLAB - 1)
            def _():
                slab = p // OUT_SLAB
                pltpu.sync_copy(
                    out_v,
                    out_hbm.at[pl.ds(base + slab * OUT_SLAB, OUT_SLAB)])


@functools.cache
def _gather_maxpool():
    return functools.partial(
        pl.kernel,
        out_type=jax.ShapeDtypeStruct((NPAIR, DP), jnp.float32),
        mesh=plsc.VectorSubcoreMesh(core_axis_name="c", subcore_axis_name="s"),
        scratch_types=[
            pltpu.VMEM((PAIRS_PER_W, LP), jnp.int32),
            pltpu.VMEM((LP, DP), jnp.float32),
            pltpu.VMEM((LP, DP), jnp.float32),
            pltpu.VMEM((OUT_SLAB, DP), jnp.float32),
            pltpu.SemaphoreType.DMA,
            pltpu.SemaphoreType.DMA,
        ],
        compiler_params=pltpu.CompilerParams(use_tc_tiling_on_sc=True),
    )(_sc_body)


def _head_body(pooled_ref, wc_ref, wr_ref, b_ref, lab_ref, loss_ref, logits_ref):
    dn = (((1,), (0,)), ((), ()))
    logits = (
        lax.dot_general(pooled_ref[0:B, 0:D], wc_ref[...], dn,
                        precision=lax.Precision.HIGHEST,
                        preferred_element_type=jnp.float32)
        + lax.dot_general(pooled_ref[B:NPAIR, 0:D], wr_ref[...], dn,
                          precision=lax.Precision.HIGHEST,
                          preferred_element_type=jnp.float32)
        + b_ref[...]
    )
    m = jnp.max(logits, axis=1, keepdims=True)
    e = jnp.exp(logits - m)
    lse = m + jnp.log(jnp.sum(e, axis=1, keepdims=True))
    picked = jnp.where(lab_ref[...] == 0, logits[:, 0:1], logits[:, 1:2])
    loss_ref[...] = jnp.sum(lse - picked, axis=0, keepdims=True) * (1.0 / B)
    logits_ref[...] = logits


_head = pl.pallas_call(
    _head_body,
    out_shape=(
        jax.ShapeDtypeStruct((1, 1), jnp.float32),
        jax.ShapeDtypeStruct((B, 2), jnp.float32),
    ),
)


def kernel(claims, reasons, label_ids, embed, W, b):
    idx = jnp.concatenate([claims, reasons], axis=0).astype(jnp.int32)
    idx = jnp.concatenate(
        [idx, jnp.broadcast_to(idx[:, :1], (NPAIR, LP - L))], axis=1)
    embed_p = _pad_table(embed.T)
    pooled = _gather_maxpool()(idx, embed_p)
    loss, logits = _head(
        pooled,
        W[:D],
        W[D:],
        b.reshape(1, 2),
        label_ids.reshape(B, 1).astype(jnp.int32),
    )
    return loss.reshape(()), logits


# speedup vs baseline: 1.8964x; 1.0168x over previous
import functools

import jax
import jax.numpy as jnp
from jax import lax
from jax.experimental import pallas as pl
from jax.experimental.pallas import tpu as pltpu
from jax.experimental.pallas import tpu_sc as plsc

B = 4096
L = 50
LP = 56
D = 300
DP = 384
NPAIR = 2 * B
NW = 32
PAIRS_PER_W = NPAIR // NW
OUT_SLAB = 128
_NCHUNK = 304 // 16

_PAD_ROWS = 1024
VOCAB = 100000


def _pad_body(in_ref, out_ref):
    y = jnp.transpose(in_ref[...], (1, 0))
    out_ref[...] = jnp.concatenate(
        [y, jnp.zeros((_PAD_ROWS, DP - D), jnp.float32)], axis=1)


_pad_table = pl.pallas_call(
    _pad_body,
    grid=((VOCAB + _PAD_ROWS - 1) // _PAD_ROWS,),
    in_specs=[pl.BlockSpec((D, _PAD_ROWS), lambda i: (0, i))],
    out_specs=pl.BlockSpec((_PAD_ROWS, DP), lambda i: (i, 0)),
    out_shape=jax.ShapeDtypeStruct((VOCAB, DP), jnp.float32),
)


def _sc_body(idx_hbm, embed_hbm, out_hbm, idx_v, rows0, rows1, out_v,
             sem0, sem1):
    nc = 2
    wid = lax.axis_index("s") * nc + lax.axis_index("c")
    base = wid * PAIRS_PER_W
    pltpu.sync_copy(idx_hbm.at[pl.ds(base, PAIRS_PER_W)], idx_v)

    bufs = (rows0, rows1)
    sems = (sem0, sem1)

    def start(p, b):
        pltpu.async_copy(embed_hbm.at[idx_v.at[p]], bufs[b], sems[b])

    start(0, 0)
    start(1, 1)

    @pl.loop(0, PAIRS_PER_W // 2)
    def _grp(g):
        for b in range(2):
            p = 2 * g + b
            rows_v = bufs[b]
            pltpu.make_async_copy(
                embed_hbm.at[idx_v.at[p]], rows_v, sems[b]).wait()

            def row_max(i, acc, rows_v=rows_v):
                return tuple(
                    jnp.maximum(a, rows_v[i, pl.ds(16 * k, 16)])
                    for k, a in enumerate(acc)
                )

            acc0 = tuple(rows_v[0, pl.ds(16 * k, 16)] for k in range(_NCHUNK))
            acc = lax.fori_loop(1, LP, row_max, acc0)
            r = lax.rem(p, OUT_SLAB)
            for k, a in enumerate(acc):
                out_v[r, pl.ds(16 * k, 16)] = a

            @pl.when(p + 2 < PAIRS_PER_W)
            def _():
                start(p + 2, b)

            @pl.when(r == OUT_SLAB - 1)
            def _():
                slab = p // OUT_SLAB
                pltpu.sync_copy(
                    out_v,
                    out_hbm.at[pl.ds(base + slab * OUT_SLAB, OUT_SLAB)])


@functools.cache
def _gather_maxpool():
    return functools.partial(
        pl.kernel,
        out_type=jax.ShapeDtypeStruct((NPAIR, DP), jnp.float32),
        mesh=plsc.VectorSubcoreMesh(core_axis_name="c", subcore_axis_name="s"),
        scratch_types=[
            pltpu.VMEM((PAIRS_PER_W, LP), jnp.int32),
            pltpu.VMEM((LP, DP), jnp.float32),
            pltpu.VMEM((LP, DP), jnp.float32),
            pltpu.VMEM((OUT_SLAB, DP), jnp.float32),
            pltpu.SemaphoreType.DMA,
            pltpu.SemaphoreType.DMA,
        ],
        compiler_params=pltpu.CompilerParams(use_tc_tiling_on_sc=True),
    )(_sc_body)


def _head_body(pooled_ref, wc_ref, wr_ref, b_ref, lab_ref, loss_ref, logits_ref):
    dn = (((1,), (0,)), ((), ()))
    logits = (
        lax.dot_general(pooled_ref[0:B, 0:D], wc_ref[...], dn,
                        preferred_element_type=jnp.float32)
        + lax.dot_general(pooled_ref[B:NPAIR, 0:D], wr_ref[...], dn,
                          preferred_element_type=jnp.float32)
        + b_ref[...]
    )
    m = jnp.max(logits, axis=1, keepdims=True)
    e = jnp.exp(logits - m)
    lse = m + jnp.log(jnp.sum(e, axis=1, keepdims=True))
    picked = jnp.where(lab_ref[...] == 0, logits[:, 0:1], logits[:, 1:2])
    loss_ref[...] = jnp.sum(lse - picked, axis=0, keepdims=True) * (1.0 / B)
    logits_ref[...] = logits


_head = pl.pallas_call(
    _head_body,
    out_shape=(
        jax.ShapeDtypeStruct((1, 1), jnp.float32),
        jax.ShapeDtypeStruct((B, 2), jnp.float32),
    ),
)


def kernel(claims, reasons, label_ids, embed, W, b):
    idx = jnp.concatenate([claims, reasons], axis=0).astype(jnp.int32)
    idx = jnp.concatenate(
        [idx, jnp.broadcast_to(idx[:, :1], (NPAIR, LP - L))], axis=1)
    embed_p = _pad_table(embed.T)
    pooled = _gather_maxpool()(idx, embed_p)
    loss, logits = _head(
        pooled,
        W[:D],
        W[D:],
        b.reshape(1, 2),
        label_ids.reshape(B, 1).astype(jnp.int32),
    )
    return loss.reshape(()), logits
